# Initial kernel scaffold; baseline (speedup 1.0000x reference)
import functools
import jax
import jax.numpy as jnp
from jax import lax
from jax.experimental import pallas as pl
from jax.experimental.pallas import tpu as pltpu
from jax.experimental.pallas import tpu_sc as plsc

N = 800000
NUM_SEG = 50000
CH = 1000


def _sc_probe(ids, feat):
    mesh = plsc.VectorSubcoreMesh(core_axis_name="c", subcore_axis_name="s",
                                  num_cores=2, num_subcores=16)

    @functools.partial(
        pl.kernel,
        out_type=jax.ShapeDtypeStruct((NUM_SEG, 32), jnp.float32),
        mesh=mesh,
        scratch_types=[
            pltpu.VMEM((CH,), jnp.int32),
            pltpu.VMEM((CH * 32,), jnp.float32),
            pltpu.VMEM((32,), jnp.float32),
        ],
    )
    def k(ids_hbm, feat_hbm, out_hbm, ids_v, feat_v, stage_v):
        wid = lax.axis_index("s") * 2 + lax.axis_index("c")
        base = wid * CH
        pltpu.sync_copy(ids_hbm.at[pl.ds(base, CH)], ids_v)
        pltpu.sync_copy(feat_hbm.at[pl.ds(base, CH), :], feat_v.reshape(CH, 32))

        def body(i, carry):
            sm0, sm1, prev = carry
            id_i = ids_v[i]
            r0 = feat_v[pl.ds(i * 32, 16)]
            r1 = feat_v[pl.ds(i * 32 + 16, 16)]
            new = id_i != prev

            @pl.when(new)
            def _():
                stage_v[pl.ds(0, 16)] = sm0
                stage_v[pl.ds(16, 16)] = sm1
                pltpu.sync_copy(stage_v, out_hbm.at[prev, :])

            sm0 = jnp.where(new, r0, sm0 + r0)
            sm1 = jnp.where(new, r1, sm1 + r1)
            return sm0, sm1, id_i

        z = jnp.zeros((16,), jnp.float32)
        lax.fori_loop(0, CH, body, (z, z, jnp.int32(-1)))

    return k(ids, feat)


def kernel(inputs, unq_inv, W, b, gamma, beta):
    feat = jnp.zeros((N, 32), jnp.float32) + inputs[:, :1]
    t = _sc_probe(unq_inv, feat)
    feat2 = t[unq_inv]
    return jnp.concatenate([feat, feat2], axis=-1)


# speedup vs baseline: 2.0941x; 2.0941x over previous
import functools

import jax
import jax.numpy as jnp
from jax import lax
from jax.experimental import pallas as pl
from jax.experimental.pallas import tpu as pltpu
from jax.experimental.pallas import tpu_sc as plsc

_N = 800000
_DIN = 10
_DH = 32
_NSEG = 50000
_EPS = 1e-3

_NW = 32
_CH = 896
_OC = 768
_BR = 8000


def _sc_mesh():
    return plsc.VectorSubcoreMesh(core_axis_name="c", subcore_axis_name="s",
                                  num_cores=2, num_subcores=16)


def _stats_body(x_ref, w_ref, b_ref, g_ref, be_ref, sc_ref, sh_ref, s1, s2):
    i = pl.program_id(0)

    @pl.when(i == 0)
    def _():
        s1[...] = jnp.zeros_like(s1)
        s2[...] = jnp.zeros_like(s2)

    x = x_ref[...]
    s1[...] += jnp.sum(x, axis=0, keepdims=True)
    s2[...] += lax.dot_general(x, x, (((0,), (0,)), ((), ())),
                               preferred_element_type=jnp.float32,
                               precision=lax.Precision.HIGHEST)

    @pl.when(i == pl.num_programs(0) - 1)
    def _():
        hi = lax.Precision.HIGHEST
        w = w_ref[...]
        m = s1[...] / _N
        c = s2[...] / _N - lax.dot_general(
            m, m, (((0,), (0,)), ((), ())),
            preferred_element_type=jnp.float32, precision=hi)
        mu = lax.dot_general(m, w, (((1,), (1,)), ((), ())),
                             preferred_element_type=jnp.float32,
                             precision=hi) + b_ref[...]
        wc = lax.dot_general(w, c, (((1,), (0,)), ((), ())),
                             preferred_element_type=jnp.float32,
                             precision=hi)
        ones = jnp.ones((1, _DIN), jnp.float32)
        var = lax.dot_general(ones, wc * w, (((1,), (1,)), ((), ())),
                              preferred_element_type=jnp.float32,
                              precision=hi)
        scale = g_ref[...] * lax.rsqrt(var + _EPS)
        sc_ref[...] = scale
        sh_ref[...] = (b_ref[...] - mu) * scale + be_ref[...]


def _fold_bn(inputs, W, b, gamma, beta):
    nb = _N // _BR
    full = pl.BlockSpec((_DH, _DIN), lambda i: (0, 0))
    row = pl.BlockSpec((1, _DH), lambda i: (0, 0))
    return pl.pallas_call(
        _stats_body,
        grid=(nb,),
        in_specs=[pl.BlockSpec((_BR, _DIN), lambda i: (i, 0)), full, row, row, row],
        out_specs=[row, row],
        out_shape=[jax.ShapeDtypeStruct((1, _DH), jnp.float32),
                   jax.ShapeDtypeStruct((1, _DH), jnp.float32)],
        scratch_shapes=[pltpu.VMEM((1, _DIN), jnp.float32),
                        pltpu.VMEM((_DIN, _DIN), jnp.float32)],
    )(inputs, W, b.reshape(1, _DH), gamma.reshape(1, _DH), beta.reshape(1, _DH))


def _feat_body(x_ref, w_ref, sc_ref, sh_ref, o_ref):
    x = x_ref[...]
    y = lax.dot_general(x, w_ref[...], (((1,), (1,)), ((), ())),
                        preferred_element_type=jnp.float32,
                        precision=lax.Precision.HIGHEST)
    o_ref[...] = jnp.maximum(y * sc_ref[...] + sh_ref[...], 0.0)


def _feat(inputs, w, scale, shift):
    nb = _N // _BR
    row = pl.BlockSpec((1, _DH), lambda i: (0, 0))
    return pl.pallas_call(
        _feat_body,
        grid=(nb,),
        in_specs=[pl.BlockSpec((_BR, _DIN), lambda i: (i, 0)),
                  pl.BlockSpec((_DH, _DIN), lambda i: (0, 0)), row, row],
        out_specs=pl.BlockSpec((_BR, _DH), lambda i: (i, 0)),
        out_shape=jax.ShapeDtypeStruct((_N, _DH), jnp.float32),
    )(inputs, w, scale, shift)


def _seg_pool(ids, feat):
    ngrp = _N // 16

    @functools.partial(
        pl.kernel,
        out_type=jax.ShapeDtypeStruct((_NSEG * _DH,), jnp.float32),
        mesh=_sc_mesh(),
        scratch_types=[
            pltpu.VMEM((_CH,), jnp.int32),
            pltpu.VMEM((_CH, _DH), jnp.float32),
            pltpu.VMEM((_DH,), jnp.float32),
            pltpu.VMEM((4, 16), jnp.float32),
        ],
    )
    def k(ids_hbm, feat_hbm, tab_hbm, ids_v, feat_v, stage_v, acc_v):
        wid = lax.axis_index("s") * 2 + lax.axis_index("c")
        rs = ((wid * ngrp) // _NW) * 16
        re = (((wid + 1) * ngrp) // _NW) * 16
        g0 = jnp.maximum(rs - 16, 0)

        zero = jnp.zeros((16,), jnp.float32)

        def emit(prev, cnt, sm0, sm1, mx0, mx1):
            inv = 1.0 / jnp.broadcast_to(cnt, (16,))
            stage_v[pl.ds(0, 16)] = 0.5 * (mx0 + sm0 * inv)
            stage_v[pl.ds(16, 16)] = 0.5 * (mx1 + sm1 * inv)
            off = pl.multiple_of(prev * _DH, 8)
            pltpu.sync_copy(stage_v, tab_hbm.at[pl.ds(off, _DH)])

        def group(gi, carry):
            done, active, prev, cnt, sm0, sm1, mx0, mx1, s = carry
            loc = gi * 16
            idv = ids_v[pl.ds(loc, 16)]
            for j in range(16):
                g_row = s + loc + j
                id_j = idv[j]
                r0 = feat_v[loc + j, pl.ds(0, 16)]
                r1 = feat_v[loc + j, pl.ds(16, 16)]
                is_new = (id_j != prev) & ~done
                close = is_new & active
                stop = is_new & (g_row >= re)

                @pl.when(close)
                def _():
                    emit(prev, cnt, sm0, sm1, mx0, mx1)

                done = done | stop
                active = (active | (is_new & (g_row >= rs))) & ~done
                sm0 = jnp.where(is_new, r0, sm0 + r0)
                sm1 = jnp.where(is_new, r1, sm1 + r1)
                mx0 = jnp.where(is_new, r0, jnp.maximum(mx0, r0))
                mx1 = jnp.where(is_new, r1, jnp.maximum(mx1, r1))
                cnt = jnp.where(is_new, 1.0, cnt + 1.0)
                prev = id_j
            return done, active, prev, cnt, sm0, sm1, mx0, mx1, s

        def chunk(_, carry):
            g, done, active, prev, cnt = carry
            live = (~done) & (g < _N)
            s = pl.multiple_of(jnp.minimum(g, _N - _CH), 16)

            @pl.when(live)
            def _():
                pltpu.sync_copy(ids_hbm.at[pl.ds(s, _CH)], ids_v)
                pltpu.sync_copy(feat_hbm.at[pl.ds(s, _CH), :], feat_v)

            lo = jnp.where(live, (g - s) // 16, _CH // 16)
            done, active, prev, cnt, sm0, sm1, mx0, mx1, _ = lax.fori_loop(
                lo, _CH // 16, group,
                (done, active, prev, cnt,
                 acc_v[0, :], acc_v[1, :], acc_v[2, :], acc_v[3, :], s))
            acc_v[0, :], acc_v[1, :] = sm0, sm1
            acc_v[2, :], acc_v[3, :] = mx0, mx1
            g = jnp.where(live, s + _CH, g)
            return g, done, active, prev, cnt

        carry = (g0, jnp.bool_(False), jnp.bool_(False), jnp.int32(-1),
                 jnp.float32(1.0))
        g, done, active, prev, cnt = lax.fori_loop(
            0, _N // _CH, chunk, carry)

        @pl.when(active & ~done)
        def _():
            emit(prev, cnt, acc_v[0, :], acc_v[1, :], acc_v[2, :], acc_v[3, :])

    return k(ids, feat)


def _gather(ids, tab):
    ngrp = _N // 16

    @functools.partial(
        pl.kernel,
        out_type=jax.ShapeDtypeStruct((_N, _DH), jnp.float32),
        mesh=_sc_mesh(),
        scratch_types=[
            pltpu.VMEM((_OC,), jnp.int32),
            pltpu.VMEM((_OC, _DH), jnp.float32),
            pltpu.VMEM((_DH,), jnp.float32),
        ],
    )
    def k(ids_hbm, tab_hbm, out_hbm, ids_v, ost_v, cur_v):
        wid = lax.axis_index("s") * 2 + lax.axis_index("c")
        ra = ((wid * ngrp) // _NW) * 16
        rb = (((wid + 1) * ngrp) // _NW) * 16
        nch = (rb - ra + _OC - 1) // _OC

        def group(gi, prev):
            loc = gi * 16
            idv = ids_v[pl.ds(loc, 16)]
            for j in range(16):
                id_j = idv[j]

                @pl.when(id_j != prev)
                def _():
                    off = pl.multiple_of(id_j * _DH, 8)
                    pltpu.sync_copy(tab_hbm.at[pl.ds(off, _DH)], cur_v)

                ost_v[loc + j, pl.ds(0, 16)] = cur_v[pl.ds(0, 16)]
                ost_v[loc + j, pl.ds(16, 16)] = cur_v[pl.ds(16, 16)]
                prev = id_j
            return prev

        def chunk(c, prev):
            s = pl.multiple_of(jnp.minimum(ra + c * _OC, rb - _OC), 16)
            pltpu.sync_copy(ids_hbm.at[pl.ds(s, _OC)], ids_v)
            prev = lax.fori_loop(0, _OC // 16, group, jnp.int32(-1))
            pltpu.sync_copy(ost_v, out_hbm.at[pl.ds(s, _OC), :])
            return prev

        lax.fori_loop(0, nch, chunk, jnp.int32(-1))

    return k(ids, tab)


def kernel(inputs, unq_inv, W, b, gamma, beta):
    scale, shift = _fold_bn(inputs, W, b, gamma, beta)
    feat = _feat(inputs, W, scale, shift)
    tab = _seg_pool(unq_inv, feat)
    out2 = _gather(unq_inv, tab)
    return jnp.concatenate([feat, out2], axis=-1)


# speedup vs baseline: 2.9317x; 1.4000x over previous
import functools

import jax
import jax.numpy as jnp
from jax import lax
from jax.experimental import pallas as pl
from jax.experimental.pallas import tpu as pltpu
from jax.experimental.pallas import tpu_sc as plsc

_N = 800000
_DIN = 10
_DH = 32
_NSEG = 50000
_EPS = 1e-3

_NW = 32
_CH = 896
_OC = 448
_TW = 512
_NR = 8
_BR = 8000


def _sc_mesh():
    return plsc.VectorSubcoreMesh(core_axis_name="c", subcore_axis_name="s",
                                  num_cores=2, num_subcores=16)


def _stats_body(x_ref, w_ref, b_ref, g_ref, be_ref, sc_ref, sh_ref, s1, s2):
    i = pl.program_id(0)

    @pl.when(i == 0)
    def _():
        s1[...] = jnp.zeros_like(s1)
        s2[...] = jnp.zeros_like(s2)

    x = x_ref[...]
    s1[...] += jnp.sum(x, axis=0, keepdims=True)
    s2[...] += lax.dot_general(x, x, (((0,), (0,)), ((), ())),
                               preferred_element_type=jnp.float32,
                               precision=lax.Precision.HIGHEST)

    @pl.when(i == pl.num_programs(0) - 1)
    def _():
        hi = lax.Precision.HIGHEST
        w = w_ref[...]
        m = s1[...] / _N
        c = s2[...] / _N - lax.dot_general(
            m, m, (((0,), (0,)), ((), ())),
            preferred_element_type=jnp.float32, precision=hi)
        mu = lax.dot_general(m, w, (((1,), (1,)), ((), ())),
                             preferred_element_type=jnp.float32,
                             precision=hi) + b_ref[...]
        wc = lax.dot_general(w, c, (((1,), (0,)), ((), ())),
                             preferred_element_type=jnp.float32,
                             precision=hi)
        ones = jnp.ones((1, _DIN), jnp.float32)
        var = lax.dot_general(ones, wc * w, (((1,), (1,)), ((), ())),
                              preferred_element_type=jnp.float32,
                              precision=hi)
        scale = g_ref[...] * lax.rsqrt(var + _EPS)
        sc_ref[...] = scale
        sh_ref[...] = (b_ref[...] - mu) * scale + be_ref[...]


def _fold_bn(inputs, W, b, gamma, beta):
    nb = _N // _BR
    full = pl.BlockSpec((_DH, _DIN), lambda i: (0, 0))
    row = pl.BlockSpec((1, _DH), lambda i: (0, 0))
    return pl.pallas_call(
        _stats_body,
        grid=(nb,),
        in_specs=[pl.BlockSpec((_BR, _DIN), lambda i: (i, 0)), full, row, row, row],
        out_specs=[row, row],
        out_shape=[jax.ShapeDtypeStruct((1, _DH), jnp.float32),
                   jax.ShapeDtypeStruct((1, _DH), jnp.float32)],
        scratch_shapes=[pltpu.VMEM((1, _DIN), jnp.float32),
                        pltpu.VMEM((_DIN, _DIN), jnp.float32)],
    )(inputs, W, b.reshape(1, _DH), gamma.reshape(1, _DH), beta.reshape(1, _DH))


def _feat_body(x_ref, w_ref, sc_ref, sh_ref, o_ref):
    x = x_ref[...]
    y = lax.dot_general(x, w_ref[...], (((1,), (1,)), ((), ())),
                        preferred_element_type=jnp.float32,
                        precision=lax.Precision.HIGHEST)
    o_ref[...] = jnp.maximum(y * sc_ref[...] + sh_ref[...], 0.0)


def _feat(inputs, w, scale, shift):
    nb = _N // _BR
    row = pl.BlockSpec((1, _DH), lambda i: (0, 0))
    return pl.pallas_call(
        _feat_body,
        grid=(nb,),
        in_specs=[pl.BlockSpec((_BR, _DIN), lambda i: (i, 0)),
                  pl.BlockSpec((_DH, _DIN), lambda i: (0, 0)), row, row],
        out_specs=pl.BlockSpec((_BR, _DH), lambda i: (i, 0)),
        out_shape=jax.ShapeDtypeStruct((_N, _DH), jnp.float32),
    )(inputs, w, scale, shift)


def _seg_pool(ids, feat):
    ngrp = _N // 16

    @functools.partial(
        pl.kernel,
        out_type=jax.ShapeDtypeStruct((_NSEG * _DH,), jnp.float32),
        mesh=_sc_mesh(),
        scratch_types=[
            pltpu.VMEM((_CH,), jnp.int32),
            pltpu.VMEM((_CH, _DH), jnp.float32),
            pltpu.VMEM((_NR * _DH,), jnp.float32),
            pltpu.VMEM((4, 16), jnp.float32),
            pltpu.SemaphoreType.DMA,
        ],
    )
    def k(ids_hbm, feat_hbm, tab_hbm, ids_v, feat_v, ring_v, acc_v, sem):
        wid = lax.axis_index("s") * 2 + lax.axis_index("c")
        rs = ((wid * ngrp) // _NW) * 16
        re = (((wid + 1) * ngrp) // _NW) * 16
        g0 = jnp.maximum(rs - 16, 0)

        zero = jnp.zeros((16,), jnp.float32)

        def emit(nclose, prev, cnt, sm0, sm1, mx0, mx1):
            slot = pl.multiple_of((nclose & (_NR - 1)) * _DH, 8)

            @pl.when(nclose >= _NR)
            def _():
                pltpu.make_async_copy(
                    tab_hbm.at[pl.ds(0, _DH)], ring_v.at[pl.ds(slot, _DH)],
                    sem).wait()

            inv = 1.0 / jnp.broadcast_to(cnt, (16,))
            ring_v[pl.ds(slot, 16)] = 0.5 * (mx0 + sm0 * inv)
            ring_v[pl.ds(slot + 16, 16)] = 0.5 * (mx1 + sm1 * inv)
            off = pl.multiple_of(prev * _DH, 8)
            pltpu.async_copy(ring_v.at[pl.ds(slot, _DH)],
                             tab_hbm.at[pl.ds(off, _DH)], sem)

        def group(gi, carry):
            done, active, prev, cnt, nclose, sm0, sm1, mx0, mx1, s = carry
            loc = gi * 16
            idv = ids_v[pl.ds(loc, 16)]
            for j in range(16):
                g_row = s + loc + j
                id_j = idv[j]
                r0 = feat_v[loc + j, pl.ds(0, 16)]
                r1 = feat_v[loc + j, pl.ds(16, 16)]
                is_new = (id_j != prev) & ~done
                close = is_new & active
                stop = is_new & (g_row >= re)

                @pl.when(close)
                def _():
                    emit(nclose, prev, cnt, sm0, sm1, mx0, mx1)

                nclose = jnp.where(close, nclose + 1, nclose)
                done = done | stop
                active = (active | (is_new & (g_row >= rs))) & ~done
                sm0 = jnp.where(is_new, r0, sm0 + r0)
                sm1 = jnp.where(is_new, r1, sm1 + r1)
                mx0 = jnp.where(is_new, r0, jnp.maximum(mx0, r0))
                mx1 = jnp.where(is_new, r1, jnp.maximum(mx1, r1))
                cnt = jnp.where(is_new, 1.0, cnt + 1.0)
                prev = id_j
            return done, active, prev, cnt, nclose, sm0, sm1, mx0, mx1, s

        def chunk(_, carry):
            g, done, active, prev, cnt, nclose = carry
            live = (~done) & (g < _N)
            s = pl.multiple_of(jnp.minimum(g, _N - _CH), 16)

            @pl.when(live)
            def _():
                pltpu.sync_copy(ids_hbm.at[pl.ds(s, _CH)], ids_v)
                pltpu.sync_copy(feat_hbm.at[pl.ds(s, _CH), :], feat_v)

            lo = jnp.where(live, (g - s) // 16, _CH // 16)
            done, active, prev, cnt, nclose, sm0, sm1, mx0, mx1, _ = \
                lax.fori_loop(
                    lo, _CH // 16, group,
                    (done, active, prev, cnt, nclose,
                     acc_v[0, :], acc_v[1, :], acc_v[2, :], acc_v[3, :], s))
            acc_v[0, :], acc_v[1, :] = sm0, sm1
            acc_v[2, :], acc_v[3, :] = mx0, mx1
            g = jnp.where(live, s + _CH, g)
            return g, done, active, prev, cnt, nclose

        carry = (g0, jnp.bool_(False), jnp.bool_(False), jnp.int32(-1),
                 jnp.float32(1.0), jnp.int32(0))
        g, done, active, prev, cnt, nclose = lax.fori_loop(
            0, _N // _CH, chunk, carry)

        @pl.when(active & ~done)
        def _():
            emit(nclose, prev, cnt,
                 acc_v[0, :], acc_v[1, :], acc_v[2, :], acc_v[3, :])

        nclose = nclose + (active & ~done).astype(jnp.int32)

        def drain(i, c):
            pltpu.make_async_copy(tab_hbm.at[pl.ds(0, _DH)],
                                  ring_v.at[pl.ds(0, _DH)], sem).wait()
            return c

        lax.fori_loop(0, jnp.minimum(nclose, _NR), drain, jnp.int32(0))

    return k(ids, feat)


def _gather(ids, tab):
    ngrp = _N // 16

    @functools.partial(
        pl.kernel,
        out_type=jax.ShapeDtypeStruct((_N, _DH), jnp.float32),
        mesh=_sc_mesh(),
        scratch_types=[
            pltpu.VMEM((_OC,), jnp.int32),
            pltpu.VMEM((_OC, _DH), jnp.float32),
            pltpu.VMEM((_TW * _DH,), jnp.float32),
            pltpu.VMEM((_DH,), jnp.float32),
        ],
    )
    def k(ids_hbm, tab_hbm, out_hbm, ids_v, ost_v, win_v, side_v):
        wid = lax.axis_index("s") * 2 + lax.axis_index("c")
        ra = ((wid * ngrp) // _NW) * 16
        rb = (((wid + 1) * ngrp) // _NW) * 16
        nch = (rb - ra + _OC - 1) // _OC

        def group(gi, wb):
            loc = gi * 16
            idv = ids_v[pl.ds(loc, 16)]
            lo, hi = idv[0], idv[15]
            trig = (hi >= wb + _TW) | (lo < wb)
            wb = jnp.where(trig, jnp.minimum(lo, _NSEG - _TW), wb)

            @pl.when(trig)
            def _():
                off = pl.multiple_of(wb * _DH, 8)
                pltpu.sync_copy(tab_hbm.at[pl.ds(off, _TW * _DH)], win_v)

            for j in range(16):
                id_j = idv[j]
                d = jnp.clip(id_j - wb, 0, _TW - 1)
                off = pl.multiple_of(d * _DH, 8)
                ost_v[loc + j, pl.ds(0, 16)] = win_v[pl.ds(off, 16)]
                ost_v[loc + j, pl.ds(16, 16)] = win_v[pl.ds(off + 16, 16)]

            @pl.when((hi - wb >= _TW) | (lo < wb))
            def _():
                for j in range(16):
                    id_j = idv[j]

                    @pl.when((id_j - wb >= _TW) | (id_j < wb))
                    def _():
                        toff = pl.multiple_of(id_j * _DH, 8)
                        pltpu.sync_copy(tab_hbm.at[pl.ds(toff, _DH)], side_v)
                        ost_v[loc + j, pl.ds(0, 16)] = side_v[pl.ds(0, 16)]
                        ost_v[loc + j, pl.ds(16, 16)] = side_v[pl.ds(16, 16)]
            return wb

        def chunk(c, wb):
            s = pl.multiple_of(jnp.minimum(ra + c * _OC, rb - _OC), 16)
            pltpu.sync_copy(ids_hbm.at[pl.ds(s, _OC)], ids_v)
            wb = lax.fori_loop(0, _OC // 16, group, wb)
            pltpu.sync_copy(ost_v, out_hbm.at[pl.ds(s, _OC), :])
            return wb

        lax.fori_loop(0, nch, chunk, jnp.int32(-_TW))

    return k(ids, tab)


def kernel(inputs, unq_inv, W, b, gamma, beta):
    scale, shift = _fold_bn(inputs, W, b, gamma, beta)
    feat = _feat(inputs, W, scale, shift)
    tab = _seg_pool(unq_inv, feat)
    out2 = _gather(unq_inv, tab)
    return jnp.concatenate([feat, out2], axis=-1)


# speedup vs baseline: 2.9332x; 1.0005x over previous
import functools

import jax
import jax.numpy as jnp
from jax import lax
from jax.experimental import pallas as pl
from jax.experimental.pallas import tpu as pltpu
from jax.experimental.pallas import tpu_sc as plsc

_N = 800000
_DIN = 10
_DH = 32
_NSEG = 50000
_EPS = 1e-3

_NW = 32
_CH = 896
_OC = 448
_TW = 512
_NR = 8
_BR = 8000
_BS = 16000


def _sc_mesh():
    return plsc.VectorSubcoreMesh(core_axis_name="c", subcore_axis_name="s",
                                  num_cores=2, num_subcores=16)


def _stats_body(x_ref, w_ref, b_ref, g_ref, be_ref, sc_ref, sh_ref, s1, s2):
    i = pl.program_id(0)

    @pl.when(i == 0)
    def _():
        s1[...] = jnp.zeros_like(s1)
        s2[...] = jnp.zeros_like(s2)

    x = x_ref[...]
    s1[...] += jnp.sum(x, axis=0, keepdims=True)
    s2[...] += lax.dot_general(x, x, (((0,), (0,)), ((), ())),
                               preferred_element_type=jnp.float32,
                               precision=lax.Precision.HIGHEST)

    @pl.when(i == pl.num_programs(0) - 1)
    def _():
        hi = lax.Precision.HIGHEST
        w = w_ref[...]
        m = s1[...] / _N
        c = s2[...] / _N - lax.dot_general(
            m, m, (((0,), (0,)), ((), ())),
            preferred_element_type=jnp.float32, precision=hi)
        mu = lax.dot_general(m, w, (((1,), (1,)), ((), ())),
                             preferred_element_type=jnp.float32,
                             precision=hi) + b_ref[...]
        wc = lax.dot_general(w, c, (((1,), (0,)), ((), ())),
                             preferred_element_type=jnp.float32,
                             precision=hi)
        ones = jnp.ones((1, _DIN), jnp.float32)
        var = lax.dot_general(ones, wc * w, (((1,), (1,)), ((), ())),
                              preferred_element_type=jnp.float32,
                              precision=hi)
        scale = g_ref[...] * lax.rsqrt(var + _EPS)
        sc_ref[...] = scale
        sh_ref[...] = (b_ref[...] - mu) * scale + be_ref[...]


def _fold_bn(inputs, W, b, gamma, beta):
    nb = _N // _BS
    full = pl.BlockSpec((_DH, _DIN), lambda i: (0, 0))
    row = pl.BlockSpec((1, _DH), lambda i: (0, 0))
    return pl.pallas_call(
        _stats_body,
        grid=(nb,),
        in_specs=[pl.BlockSpec((_BS, _DIN), lambda i: (i, 0)), full, row, row, row],
        out_specs=[row, row],
        out_shape=[jax.ShapeDtypeStruct((1, _DH), jnp.float32),
                   jax.ShapeDtypeStruct((1, _DH), jnp.float32)],
        scratch_shapes=[pltpu.VMEM((1, _DIN), jnp.float32),
                        pltpu.VMEM((_DIN, _DIN), jnp.float32)],
    )(inputs, W, b.reshape(1, _DH), gamma.reshape(1, _DH), beta.reshape(1, _DH))


def _feat_body(x_ref, w_ref, sc_ref, sh_ref, o_ref):
    x = x_ref[...]
    y = lax.dot_general(x, w_ref[...], (((1,), (1,)), ((), ())),
                        preferred_element_type=jnp.float32,
                        precision=lax.Precision.HIGHEST)
    o_ref[...] = jnp.maximum(y * sc_ref[...] + sh_ref[...], 0.0)


def _feat(inputs, w, scale, shift):
    nb = _N // _BR
    row = pl.BlockSpec((1, _DH), lambda i: (0, 0))
    return pl.pallas_call(
        _feat_body,
        grid=(nb,),
        in_specs=[pl.BlockSpec((_BR, _DIN), lambda i: (i, 0)),
                  pl.BlockSpec((_DH, _DIN), lambda i: (0, 0)), row, row],
        out_specs=pl.BlockSpec((_BR, _DH), lambda i: (i, 0)),
        out_shape=jax.ShapeDtypeStruct((_N, _DH), jnp.float32),
    )(inputs, w, scale, shift)


def _seg_pool(ids, feat):
    ngrp = _N // 16

    @functools.partial(
        pl.kernel,
        out_type=jax.ShapeDtypeStruct((_NSEG * _DH,), jnp.float32),
        mesh=_sc_mesh(),
        scratch_types=[
            pltpu.VMEM((_CH,), jnp.int32),
            pltpu.VMEM((_CH, _DH), jnp.float32),
            pltpu.VMEM((_NR * _DH,), jnp.float32),
            pltpu.VMEM((4, 16), jnp.float32),
            pltpu.SemaphoreType.DMA,
        ],
    )
    def k(ids_hbm, feat_hbm, tab_hbm, ids_v, feat_v, ring_v, acc_v, sem):
        wid = lax.axis_index("s") * 2 + lax.axis_index("c")
        rs = ((wid * ngrp) // _NW) * 16
        re = (((wid + 1) * ngrp) // _NW) * 16
        g0 = jnp.maximum(rs - 16, 0)

        zero = jnp.zeros((16,), jnp.float32)

        def emit(nclose, prev, cnt, sm0, sm1, mx0, mx1):
            slot = pl.multiple_of((nclose & (_NR - 1)) * _DH, 8)

            @pl.when(nclose >= _NR)
            def _():
                pltpu.make_async_copy(
                    tab_hbm.at[pl.ds(0, _DH)], ring_v.at[pl.ds(slot, _DH)],
                    sem).wait()

            inv = 1.0 / jnp.broadcast_to(cnt, (16,))
            ring_v[pl.ds(slot, 16)] = 0.5 * (mx0 + sm0 * inv)
            ring_v[pl.ds(slot + 16, 16)] = 0.5 * (mx1 + sm1 * inv)
            off = pl.multiple_of(prev * _DH, 8)
            pltpu.async_copy(ring_v.at[pl.ds(slot, _DH)],
                             tab_hbm.at[pl.ds(off, _DH)], sem)

        def group(gi, carry):
            done, active, prev, cnt, nclose, sm0, sm1, mx0, mx1, s = carry
            loc = gi * 16
            idv = ids_v[pl.ds(loc, 16)]
            for j in range(16):
                g_row = s + loc + j
                id_j = idv[j]
                r0 = feat_v[loc + j, pl.ds(0, 16)]
                r1 = feat_v[loc + j, pl.ds(16, 16)]
                is_new = (id_j != prev) & ~done
                close = is_new & active
                stop = is_new & (g_row >= re)

                @pl.when(close)
                def _():
                    emit(nclose, prev, cnt, sm0, sm1, mx0, mx1)

                nclose = jnp.where(close, nclose + 1, nclose)
                done = done | stop
                active = (active | (is_new & (g_row >= rs))) & ~done
                sm0 = jnp.where(is_new, r0, sm0 + r0)
                sm1 = jnp.where(is_new, r1, sm1 + r1)
                mx0 = jnp.where(is_new, r0, jnp.maximum(mx0, r0))
                mx1 = jnp.where(is_new, r1, jnp.maximum(mx1, r1))
                cnt = jnp.where(is_new, 1.0, cnt + 1.0)
                prev = id_j
            return done, active, prev, cnt, nclose, sm0, sm1, mx0, mx1, s

        def chunk(_, carry):
            g, done, active, prev, cnt, nclose = carry
            live = (~done) & (g < _N)
            s = pl.multiple_of(jnp.minimum(g, _N - _CH), 16)

            @pl.when(live)
            def _():
                pltpu.sync_copy(ids_hbm.at[pl.ds(s, _CH)], ids_v)
                pltpu.sync_copy(feat_hbm.at[pl.ds(s, _CH), :], feat_v)

            lo = jnp.where(live, (g - s) // 16, _CH // 16)
            done, active, prev, cnt, nclose, sm0, sm1, mx0, mx1, _ = \
                lax.fori_loop(
                    lo, _CH // 16, group,
                    (done, active, prev, cnt, nclose,
                     acc_v[0, :], acc_v[1, :], acc_v[2, :], acc_v[3, :], s))
            acc_v[0, :], acc_v[1, :] = sm0, sm1
            acc_v[2, :], acc_v[3, :] = mx0, mx1
            g = jnp.where(live, s + _CH, g)
            return g, done, active, prev, cnt, nclose

        carry = (g0, jnp.bool_(False), jnp.bool_(False), jnp.int32(-1),
                 jnp.float32(1.0), jnp.int32(0))
        g, done, active, prev, cnt, nclose = lax.fori_loop(
            0, _N // _CH, chunk, carry)

        @pl.when(active & ~done)
        def _():
            emit(nclose, prev, cnt,
                 acc_v[0, :], acc_v[1, :], acc_v[2, :], acc_v[3, :])

        nclose = nclose + (active & ~done).astype(jnp.int32)

        def drain(i, c):
            pltpu.make_async_copy(tab_hbm.at[pl.ds(0, _DH)],
                                  ring_v.at[pl.ds(0, _DH)], sem).wait()
            return c

        lax.fori_loop(0, jnp.minimum(nclose, _NR), drain, jnp.int32(0))

    return k(ids, feat)


def _gather(ids, tab):
    ngrp = _N // 16

    @functools.partial(
        pl.kernel,
        out_type=jax.ShapeDtypeStruct((_N, _DH), jnp.float32),
        mesh=_sc_mesh(),
        scratch_types=[
            pltpu.VMEM((_OC,), jnp.int32),
            pltpu.VMEM((_OC, _DH), jnp.float32),
            pltpu.VMEM((_TW * _DH,), jnp.float32),
            pltpu.VMEM((_DH,), jnp.float32),
        ],
    )
    def k(ids_hbm, tab_hbm, out_hbm, ids_v, ost_v, win_v, side_v):
        wid = lax.axis_index("s") * 2 + lax.axis_index("c")
        ra = ((wid * ngrp) // _NW) * 16
        rb = (((wid + 1) * ngrp) // _NW) * 16
        nch = (rb - ra + _OC - 1) // _OC

        def group(gi, wb):
            loc = gi * 16
            idv = ids_v[pl.ds(loc, 16)]
            lo, hi = idv[0], idv[15]
            trig = (hi >= wb + _TW) | (lo < wb)
            wb = jnp.where(trig, jnp.minimum(lo, _NSEG - _TW), wb)

            @pl.when(trig)
            def _():
                off = pl.multiple_of(wb * _DH, 8)
                pltpu.sync_copy(tab_hbm.at[pl.ds(off, _TW * _DH)], win_v)

            for j in range(16):
                id_j = idv[j]
                d = jnp.clip(id_j - wb, 0, _TW - 1)
                off = pl.multiple_of(d * _DH, 8)
                ost_v[loc + j, pl.ds(0, 16)] = win_v[pl.ds(off, 16)]
                ost_v[loc + j, pl.ds(16, 16)] = win_v[pl.ds(off + 16, 16)]

            @pl.when((hi - wb >= _TW) | (lo < wb))
            def _():
                for j in range(16):
                    id_j = idv[j]

                    @pl.when((id_j - wb >= _TW) | (id_j < wb))
                    def _():
                        toff = pl.multiple_of(id_j * _DH, 8)
                        pltpu.sync_copy(tab_hbm.at[pl.ds(toff, _DH)], side_v)
                        ost_v[loc + j, pl.ds(0, 16)] = side_v[pl.ds(0, 16)]
                        ost_v[loc + j, pl.ds(16, 16)] = side_v[pl.ds(16, 16)]
            return wb

        def chunk(c, wb):
            s = pl.multiple_of(jnp.minimum(ra + c * _OC, rb - _OC), 16)
            pltpu.sync_copy(ids_hbm.at[pl.ds(s, _OC)], ids_v)
            wb = lax.fori_loop(0, _OC // 16, group, wb)
            pltpu.sync_copy(ost_v, out_hbm.at[pl.ds(s, _OC), :])
            return wb

        lax.fori_loop(0, nch, chunk, jnp.int32(-_TW))

    return k(ids, tab)


def kernel(inputs, unq_inv, W, b, gamma, beta):
    scale, shift = _fold_bn(inputs, W, b, gamma, beta)
    feat = _feat(inputs, W, scale, shift)
    tab = _seg_pool(unq_inv, feat)
    out2 = _gather(unq_inv, tab)
    return jnp.concatenate([feat, out2], axis=-1)


# speedup vs baseline: 2.9367x; 1.0012x over previous
import functools

import jax
import jax.numpy as jnp
from jax import lax
from jax.experimental import pallas as pl
from jax.experimental.pallas import tpu as pltpu
from jax.experimental.pallas import tpu_sc as plsc

_N = 800000
_DIN = 10
_DH = 32
_NSEG = 50000
_EPS = 1e-3

_NW = 32
_CH = 896
_OC = 448
_TW = 512
_NR = 8
_BR = 8000
_BS = 16000


def _sc_mesh():
    return plsc.VectorSubcoreMesh(core_axis_name="c", subcore_axis_name="s",
                                  num_cores=2, num_subcores=16)


def _stats_body(x_ref, w_ref, b_ref, g_ref, be_ref, sc_ref, sh_ref, s1, s2):
    i = pl.program_id(0)

    @pl.when(i == 0)
    def _():
        s1[...] = jnp.zeros_like(s1)
        s2[...] = jnp.zeros_like(s2)

    x = x_ref[...]
    s1[...] += jnp.sum(x, axis=0, keepdims=True)
    s2[...] += lax.dot_general(x, x, (((0,), (0,)), ((), ())),
                               preferred_element_type=jnp.float32,
                               precision=lax.Precision.HIGHEST)

    @pl.when(i == pl.num_programs(0) - 1)
    def _():
        hi = lax.Precision.HIGHEST
        w = w_ref[...]
        m = s1[...] / _N
        c = s2[...] / _N - lax.dot_general(
            m, m, (((0,), (0,)), ((), ())),
            preferred_element_type=jnp.float32, precision=hi)
        mu = lax.dot_general(m, w, (((1,), (1,)), ((), ())),
                             preferred_element_type=jnp.float32,
                             precision=hi) + b_ref[...]
        wc = lax.dot_general(w, c, (((1,), (0,)), ((), ())),
                             preferred_element_type=jnp.float32,
                             precision=hi)
        ones = jnp.ones((1, _DIN), jnp.float32)
        var = lax.dot_general(ones, wc * w, (((1,), (1,)), ((), ())),
                              preferred_element_type=jnp.float32,
                              precision=hi)
        scale = g_ref[...] * lax.rsqrt(var + _EPS)
        sc_ref[...] = scale
        sh_ref[...] = (b_ref[...] - mu) * scale + be_ref[...]


def _fold_bn(inputs, W, b, gamma, beta):
    nb = _N // _BS
    full = pl.BlockSpec((_DH, _DIN), lambda i: (0, 0))
    row = pl.BlockSpec((1, _DH), lambda i: (0, 0))
    return pl.pallas_call(
        _stats_body,
        grid=(nb,),
        in_specs=[pl.BlockSpec((_BS, _DIN), lambda i: (i, 0)), full, row, row, row],
        out_specs=[row, row],
        out_shape=[jax.ShapeDtypeStruct((1, _DH), jnp.float32),
                   jax.ShapeDtypeStruct((1, _DH), jnp.float32)],
        scratch_shapes=[pltpu.VMEM((1, _DIN), jnp.float32),
                        pltpu.VMEM((_DIN, _DIN), jnp.float32)],
    )(inputs, W, b.reshape(1, _DH), gamma.reshape(1, _DH), beta.reshape(1, _DH))


def _feat_body(x_ref, w_ref, sc_ref, sh_ref, o_ref):
    x = x_ref[...]
    y = lax.dot_general(x, w_ref[...], (((1,), (1,)), ((), ())),
                        preferred_element_type=jnp.float32,
                        precision=lax.Precision.HIGHEST)
    o_ref[...] = jnp.maximum(y * sc_ref[...] + sh_ref[...], 0.0)


def _feat(inputs, w, scale, shift):
    nb = _N // _BR
    row = pl.BlockSpec((1, _DH), lambda i: (0, 0))
    return pl.pallas_call(
        _feat_body,
        grid=(nb,),
        in_specs=[pl.BlockSpec((_BR, _DIN), lambda i: (i, 0)),
                  pl.BlockSpec((_DH, _DIN), lambda i: (0, 0)), row, row],
        out_specs=pl.BlockSpec((_BR, _DH), lambda i: (i, 0)),
        out_shape=jax.ShapeDtypeStruct((_N, _DH), jnp.float32),
    )(inputs, w, scale, shift)


def _seg_pool(ids, feat):
    ngrp = _N // 16

    @functools.partial(
        pl.kernel,
        out_type=jax.ShapeDtypeStruct((_NSEG * _DH,), jnp.float32),
        mesh=_sc_mesh(),
        scratch_types=[
            pltpu.VMEM((_CH,), jnp.int32),
            pltpu.VMEM((_CH, _DH), jnp.float32),
            pltpu.VMEM((_NR * _DH,), jnp.float32),
            pltpu.VMEM((4, 16), jnp.float32),
            pltpu.SemaphoreType.DMA,
        ],
    )
    def k(ids_hbm, feat_hbm, tab_hbm, ids_v, feat_v, ring_v, acc_v, sem):
        wid = lax.axis_index("s") * 2 + lax.axis_index("c")
        rs = ((wid * ngrp) // _NW) * 16
        re = (((wid + 1) * ngrp) // _NW) * 16
        g0 = jnp.maximum(rs - 16, 0)

        zero = jnp.zeros((16,), jnp.float32)

        def emit(nclose, prev, cnt, sm0, sm1, mx0, mx1):
            slot = pl.multiple_of((nclose & (_NR - 1)) * _DH, 8)

            @pl.when(nclose >= _NR)
            def _():
                pltpu.make_async_copy(
                    tab_hbm.at[pl.ds(0, _DH)], ring_v.at[pl.ds(slot, _DH)],
                    sem).wait()

            inv = 1.0 / jnp.broadcast_to(cnt, (16,))
            ring_v[pl.ds(slot, 16)] = 0.5 * (mx0 + sm0 * inv)
            ring_v[pl.ds(slot + 16, 16)] = 0.5 * (mx1 + sm1 * inv)
            off = pl.multiple_of(prev * _DH, 8)
            pltpu.async_copy(ring_v.at[pl.ds(slot, _DH)],
                             tab_hbm.at[pl.ds(off, _DH)], sem)

        def group(gi, carry):
            done, active, prev, cnt, nclose, sm0, sm1, mx0, mx1, s = carry
            loc = gi * 16
            idv = ids_v[pl.ds(loc, 16)]
            for j in range(16):
                g_row = s + loc + j
                id_j = idv[j]
                r0 = feat_v[loc + j, pl.ds(0, 16)]
                r1 = feat_v[loc + j, pl.ds(16, 16)]
                is_new = (id_j != prev) & ~done
                close = is_new & active
                stop = is_new & (g_row >= re)

                @pl.when(close)
                def _():
                    emit(nclose, prev, cnt, sm0, sm1, mx0, mx1)

                nclose = jnp.where(close, nclose + 1, nclose)
                done = done | stop
                active = (active | (is_new & (g_row >= rs))) & ~done
                sm0 = jnp.where(is_new, r0, sm0 + r0)
                sm1 = jnp.where(is_new, r1, sm1 + r1)
                mx0 = jnp.where(is_new, r0, jnp.maximum(mx0, r0))
                mx1 = jnp.where(is_new, r1, jnp.maximum(mx1, r1))
                cnt = jnp.where(is_new, 1.0, cnt + 1.0)
                prev = id_j
            return done, active, prev, cnt, nclose, sm0, sm1, mx0, mx1, s

        def group_mid(gi, carry):
            active, prev, cnt, nclose, sm0, sm1, mx0, mx1, s = carry
            loc = gi * 16
            idv = ids_v[pl.ds(loc, 16)]
            for j in range(16):
                id_j = idv[j]
                r0 = feat_v[loc + j, pl.ds(0, 16)]
                r1 = feat_v[loc + j, pl.ds(16, 16)]
                is_new = id_j != prev
                close = is_new & active

                @pl.when(close)
                def _():
                    emit(nclose, prev, cnt, sm0, sm1, mx0, mx1)

                nclose = jnp.where(close, nclose + 1, nclose)
                active = active | (is_new & ((s + loc + j) >= rs))
                sm0 = jnp.where(is_new, r0, sm0 + r0)
                sm1 = jnp.where(is_new, r1, sm1 + r1)
                mx0 = jnp.where(is_new, r0, jnp.maximum(mx0, r0))
                mx1 = jnp.where(is_new, r1, jnp.maximum(mx1, r1))
                cnt = jnp.where(is_new, 1.0, cnt + 1.0)
                prev = id_j
            return active, prev, cnt, nclose, sm0, sm1, mx0, mx1, s

        def chunk_mid(k, carry):
            active, prev, cnt, nclose = carry
            s = pl.multiple_of(g0 + k * _CH, 16)
            pltpu.sync_copy(ids_hbm.at[pl.ds(s, _CH)], ids_v)
            pltpu.sync_copy(feat_hbm.at[pl.ds(s, _CH), :], feat_v)
            active, prev, cnt, nclose, sm0, sm1, mx0, mx1, _ = lax.fori_loop(
                0, _CH // 16, group_mid,
                (active, prev, cnt, nclose,
                 acc_v[0, :], acc_v[1, :], acc_v[2, :], acc_v[3, :], s))
            acc_v[0, :], acc_v[1, :] = sm0, sm1
            acc_v[2, :], acc_v[3, :] = mx0, mx1
            return active, prev, cnt, nclose

        def chunk(_, carry):
            g, done, active, prev, cnt, nclose = carry
            live = (~done) & (g < _N)
            s = pl.multiple_of(jnp.minimum(g, _N - _CH), 16)

            @pl.when(live)
            def _():
                pltpu.sync_copy(ids_hbm.at[pl.ds(s, _CH)], ids_v)
                pltpu.sync_copy(feat_hbm.at[pl.ds(s, _CH), :], feat_v)

            lo = jnp.where(live, (g - s) // 16, _CH // 16)
            done, active, prev, cnt, nclose, sm0, sm1, mx0, mx1, _ = \
                lax.fori_loop(
                    lo, _CH // 16, group,
                    (done, active, prev, cnt, nclose,
                     acc_v[0, :], acc_v[1, :], acc_v[2, :], acc_v[3, :], s))
            acc_v[0, :], acc_v[1, :] = sm0, sm1
            acc_v[2, :], acc_v[3, :] = mx0, mx1
            g = jnp.where(live, s + _CH, g)
            return g, done, active, prev, cnt, nclose

        n_mid = (re - g0) // _CH
        active, prev, cnt, nclose = lax.fori_loop(
            0, n_mid, chunk_mid,
            (jnp.bool_(False), jnp.int32(-1), jnp.float32(1.0), jnp.int32(0)))

        carry = (g0 + n_mid * _CH, jnp.bool_(False), active, prev, cnt, nclose)
        g, done, active, prev, cnt, nclose = lax.fori_loop(
            0, _N // _CH, chunk, carry)

        @pl.when(active & ~done)
        def _():
            emit(nclose, prev, cnt,
                 acc_v[0, :], acc_v[1, :], acc_v[2, :], acc_v[3, :])

        nclose = nclose + (active & ~done).astype(jnp.int32)

        def drain(i, c):
            pltpu.make_async_copy(tab_hbm.at[pl.ds(0, _DH)],
                                  ring_v.at[pl.ds(0, _DH)], sem).wait()
            return c

        lax.fori_loop(0, jnp.minimum(nclose, _NR), drain, jnp.int32(0))

    return k(ids, feat)


def _gather(ids, tab):
    ngrp = _N // 16

    @functools.partial(
        pl.kernel,
        out_type=jax.ShapeDtypeStruct((_N, _DH), jnp.float32),
        mesh=_sc_mesh(),
        scratch_types=[
            pltpu.VMEM((_OC,), jnp.int32),
            pltpu.VMEM((_OC, _DH), jnp.float32),
            pltpu.VMEM((_TW * _DH,), jnp.float32),
            pltpu.VMEM((_DH,), jnp.float32),
        ],
    )
    def k(ids_hbm, tab_hbm, out_hbm, ids_v, ost_v, win_v, side_v):
        wid = lax.axis_index("s") * 2 + lax.axis_index("c")
        ra = ((wid * ngrp) // _NW) * 16
        rb = (((wid + 1) * ngrp) // _NW) * 16
        nch = (rb - ra + _OC - 1) // _OC

        def group(gi, wb):
            loc = gi * 16
            idv = ids_v[pl.ds(loc, 16)]
            lo, hi = idv[0], idv[15]
            trig = (hi >= wb + _TW) | (lo < wb)
            wb = jnp.where(trig, jnp.minimum(lo, _NSEG - _TW), wb)

            @pl.when(trig)
            def _():
                off = pl.multiple_of(wb * _DH, 8)
                pltpu.sync_copy(tab_hbm.at[pl.ds(off, _TW * _DH)], win_v)

            for j in range(16):
                id_j = idv[j]
                d = jnp.clip(id_j - wb, 0, _TW - 1)
                off = pl.multiple_of(d * _DH, 8)
                ost_v[loc + j, pl.ds(0, 16)] = win_v[pl.ds(off, 16)]
                ost_v[loc + j, pl.ds(16, 16)] = win_v[pl.ds(off + 16, 16)]

            @pl.when((hi - wb >= _TW) | (lo < wb))
            def _():
                for j in range(16):
                    id_j = idv[j]

                    @pl.when((id_j - wb >= _TW) | (id_j < wb))
                    def _():
                        toff = pl.multiple_of(id_j * _DH, 8)
                        pltpu.sync_copy(tab_hbm.at[pl.ds(toff, _DH)], side_v)
                        ost_v[loc + j, pl.ds(0, 16)] = side_v[pl.ds(0, 16)]
                        ost_v[loc + j, pl.ds(16, 16)] = side_v[pl.ds(16, 16)]
            return wb

        def chunk(c, wb):
            s = pl.multiple_of(jnp.minimum(ra + c * _OC, rb - _OC), 16)
            pltpu.sync_copy(ids_hbm.at[pl.ds(s, _OC)], ids_v)
            wb = lax.fori_loop(0, _OC // 16, group, wb)
            pltpu.sync_copy(ost_v, out_hbm.at[pl.ds(s, _OC), :])
            return wb

        lax.fori_loop(0, nch, chunk, jnp.int32(-_TW))

    return k(ids, tab)


def kernel(inputs, unq_inv, W, b, gamma, beta):
    scale, shift = _fold_bn(inputs, W, b, gamma, beta)
    feat = _feat(inputs, W, scale, shift)
    tab = _seg_pool(unq_inv, feat)
    out2 = _gather(unq_inv, tab)
    return jnp.concatenate([feat, out2], axis=-1)


# speedup vs baseline: 3.2660x; 1.1121x over previous
import functools

import jax
import jax.numpy as jnp
from jax import lax
from jax.experimental import pallas as pl
from jax.experimental.pallas import tpu as pltpu
from jax.experimental.pallas import tpu_sc as plsc

_N = 800000
_DIN = 10
_DH = 32
_NSEG = 50000
_EPS = 1e-3

_NW = 32
_CH = 896
_OC = 448
_TW = 512
_NR = 8
_BR = 8000
_BS = 16000


def _sc_mesh():
    return plsc.VectorSubcoreMesh(core_axis_name="c", subcore_axis_name="s",
                                  num_cores=2, num_subcores=16)


def _stats_body(x_ref, w_ref, b_ref, g_ref, be_ref, sc_ref, sh_ref, s1, s2):
    i = pl.program_id(0)

    @pl.when(i == 0)
    def _():
        s1[...] = jnp.zeros_like(s1)
        s2[...] = jnp.zeros_like(s2)

    x = x_ref[...]
    s1[...] += jnp.sum(x, axis=0, keepdims=True)
    s2[...] += lax.dot_general(x, x, (((0,), (0,)), ((), ())),
                               preferred_element_type=jnp.float32,
                               precision=lax.Precision.HIGHEST)

    @pl.when(i == pl.num_programs(0) - 1)
    def _():
        hi = lax.Precision.HIGHEST
        w = w_ref[...]
        m = s1[...] / _N
        c = s2[...] / _N - lax.dot_general(
            m, m, (((0,), (0,)), ((), ())),
            preferred_element_type=jnp.float32, precision=hi)
        mu = lax.dot_general(m, w, (((1,), (1,)), ((), ())),
                             preferred_element_type=jnp.float32,
                             precision=hi) + b_ref[...]
        wc = lax.dot_general(w, c, (((1,), (0,)), ((), ())),
                             preferred_element_type=jnp.float32,
                             precision=hi)
        ones = jnp.ones((1, _DIN), jnp.float32)
        var = lax.dot_general(ones, wc * w, (((1,), (1,)), ((), ())),
                              preferred_element_type=jnp.float32,
                              precision=hi)
        scale = g_ref[...] * lax.rsqrt(var + _EPS)
        sc_ref[...] = scale
        sh_ref[...] = (b_ref[...] - mu) * scale + be_ref[...]


def _fold_bn(inputs, W, b, gamma, beta):
    nb = _N // _BS
    full = pl.BlockSpec((_DH, _DIN), lambda i: (0, 0))
    row = pl.BlockSpec((1, _DH), lambda i: (0, 0))
    return pl.pallas_call(
        _stats_body,
        grid=(nb,),
        in_specs=[pl.BlockSpec((_BS, _DIN), lambda i: (i, 0)), full, row, row, row],
        out_specs=[row, row],
        out_shape=[jax.ShapeDtypeStruct((1, _DH), jnp.float32),
                   jax.ShapeDtypeStruct((1, _DH), jnp.float32)],
        scratch_shapes=[pltpu.VMEM((1, _DIN), jnp.float32),
                        pltpu.VMEM((_DIN, _DIN), jnp.float32)],
    )(inputs, W, b.reshape(1, _DH), gamma.reshape(1, _DH), beta.reshape(1, _DH))


def _feat_body(x_ref, w_ref, sc_ref, sh_ref, o_ref):
    x = x_ref[...]
    y = lax.dot_general(x, w_ref[...], (((1,), (1,)), ((), ())),
                        preferred_element_type=jnp.float32,
                        precision=lax.Precision.HIGHEST)
    o_ref[...] = jnp.maximum(y * sc_ref[...] + sh_ref[...], 0.0)


def _feat(inputs, w, scale, shift):
    nb = _N // _BR
    row = pl.BlockSpec((1, _DH), lambda i: (0, 0))
    return pl.pallas_call(
        _feat_body,
        grid=(nb,),
        in_specs=[pl.BlockSpec((_BR, _DIN), lambda i: (i, 0)),
                  pl.BlockSpec((_DH, _DIN), lambda i: (0, 0)), row, row],
        out_specs=pl.BlockSpec((_BR, _DH), lambda i: (i, 0)),
        out_shape=jax.ShapeDtypeStruct((_N, _DH), jnp.float32),
    )(inputs, w, scale, shift)


def _seg_pool(ids, feat):
    ngrp = _N // 16

    @functools.partial(
        pl.kernel,
        out_type=jax.ShapeDtypeStruct((_NSEG * _DH,), jnp.float32),
        mesh=_sc_mesh(),
        scratch_types=[
            pltpu.VMEM((_CH,), jnp.int32),
            pltpu.VMEM((_CH, _DH), jnp.float32),
            pltpu.VMEM((_NR * _DH,), jnp.float32),
            pltpu.VMEM((5, 16), jnp.float32),
            pltpu.VMEM((16 * 96,), jnp.float32),
            pltpu.SemaphoreType.DMA,
        ],
    )
    def k(ids_hbm, feat_hbm, tab_hbm, ids_v, feat_v, ring_v, acc_v, pend_v,
          sem):
        wid = lax.axis_index("s") * 2 + lax.axis_index("c")
        rs = ((wid * ngrp) // _NW) * 16
        re = (((wid + 1) * ngrp) // _NW) * 16
        g0 = jnp.maximum(rs - 16, 0)

        zero = jnp.zeros((16,), jnp.float32)

        def emit(nclose, prev, cnt, sm0, sm1, mx0, mx1):
            slot = pl.multiple_of((nclose & (_NR - 1)) * _DH, 8)

            @pl.when(nclose >= _NR)
            def _():
                pltpu.make_async_copy(
                    tab_hbm.at[pl.ds(0, _DH)], ring_v.at[pl.ds(slot, _DH)],
                    sem).wait()

            inv = 1.0 / cnt
            ring_v[pl.ds(slot, 16)] = 0.5 * (mx0 + sm0 * inv)
            ring_v[pl.ds(slot + 16, 16)] = 0.5 * (mx1 + sm1 * inv)
            off = pl.multiple_of(prev * _DH, 8)
            pltpu.async_copy(ring_v.at[pl.ds(slot, _DH)],
                             tab_hbm.at[pl.ds(off, _DH)], sem)

        def group(gi, carry):
            done, active, prev, cnt, nclose, sm0, sm1, mx0, mx1, s = carry
            loc = gi * 16
            idv = ids_v[pl.ds(loc, 16)]
            for j in range(16):
                g_row = s + loc + j
                id_j = idv[j]
                r0 = feat_v[loc + j, pl.ds(0, 16)]
                r1 = feat_v[loc + j, pl.ds(16, 16)]
                is_new = (id_j != prev) & ~done
                close = is_new & active
                stop = is_new & (g_row >= re)

                @pl.when(close)
                def _():
                    emit(nclose, prev, cnt, sm0, sm1, mx0, mx1)

                nclose = jnp.where(close, nclose + 1, nclose)
                done = done | stop
                active = (active | (is_new & (g_row >= rs))) & ~done
                sm0 = jnp.where(is_new, r0, sm0 + r0)
                sm1 = jnp.where(is_new, r1, sm1 + r1)
                mx0 = jnp.where(is_new, r0, jnp.maximum(mx0, r0))
                mx1 = jnp.where(is_new, r1, jnp.maximum(mx1, r1))
                cnt = jnp.where(is_new, 1.0, cnt + 1.0)
                prev = id_j
            return done, active, prev, cnt, nclose, sm0, sm1, mx0, mx1, s

        def group_mid(gi, carry):
            active, prev, cnt, nclose, sm0, sm1, mx0, mx1, s = carry
            loc = gi * 16
            idv = ids_v[pl.ds(loc, 16)]
            pc = jnp.int32(0)
            for j in range(16):
                id_j = idv[j]
                r0 = feat_v[loc + j, pl.ds(0, 16)]
                r1 = feat_v[loc + j, pl.ds(16, 16)]
                is_new = id_j != prev
                close = is_new & active

                @pl.when(close)
                def _():
                    base = pc * 96
                    pend_v[pl.ds(base, 16)] = sm0
                    pend_v[pl.ds(base + 16, 16)] = sm1
                    pend_v[pl.ds(base + 32, 16)] = mx0
                    pend_v[pl.ds(base + 48, 16)] = mx1
                    pend_v[pl.ds(base + 64, 16)] = cnt
                    pend_v[pl.ds(base + 80, 16)] = jnp.broadcast_to(
                        prev.astype(jnp.float32), (16,))

                pc = jnp.where(close, pc + 1, pc)
                active = active | (is_new & ((s + loc + j) >= rs))
                sm0 = jnp.where(is_new, r0, sm0 + r0)
                sm1 = jnp.where(is_new, r1, sm1 + r1)
                mx0 = jnp.where(is_new, r0, jnp.maximum(mx0, r0))
                mx1 = jnp.where(is_new, r1, jnp.maximum(mx1, r1))
                cnt = jnp.where(is_new, 1.0, cnt + 1.0)
                prev = id_j

            def drain_pend(i, ncl):
                b = i * 96
                pid = pend_v[pl.ds(b + 80, 16)][0].astype(jnp.int32)
                emit(ncl, pid, pend_v[pl.ds(b + 64, 16)],
                     pend_v[pl.ds(b, 16)], pend_v[pl.ds(b + 16, 16)],
                     pend_v[pl.ds(b + 32, 16)], pend_v[pl.ds(b + 48, 16)])
                return ncl + 1

            nclose = lax.fori_loop(0, pc, drain_pend, nclose)
            return active, prev, cnt, nclose, sm0, sm1, mx0, mx1, s

        def chunk_mid(k, carry):
            active, prev, nclose = carry
            s = pl.multiple_of(g0 + k * _CH, 16)
            pltpu.sync_copy(ids_hbm.at[pl.ds(s, _CH)], ids_v)
            pltpu.sync_copy(feat_hbm.at[pl.ds(s, _CH), :], feat_v)
            active, prev, cnt, nclose, sm0, sm1, mx0, mx1, _ = lax.fori_loop(
                0, _CH // 16, group_mid,
                (active, prev, acc_v[4, :], nclose,
                 acc_v[0, :], acc_v[1, :], acc_v[2, :], acc_v[3, :], s))
            acc_v[0, :], acc_v[1, :] = sm0, sm1
            acc_v[2, :], acc_v[3, :] = mx0, mx1
            acc_v[4, :] = cnt
            return active, prev, nclose

        def chunk(_, carry):
            g, done, active, prev, nclose = carry
            live = (~done) & (g < _N)
            s = pl.multiple_of(jnp.minimum(g, _N - _CH), 16)

            @pl.when(live)
            def _():
                pltpu.sync_copy(ids_hbm.at[pl.ds(s, _CH)], ids_v)
                pltpu.sync_copy(feat_hbm.at[pl.ds(s, _CH), :], feat_v)

            lo = jnp.where(live, (g - s) // 16, _CH // 16)
            done, active, prev, cnt, nclose, sm0, sm1, mx0, mx1, _ = \
                lax.fori_loop(
                    lo, _CH // 16, group,
                    (done, active, prev, acc_v[4, :], nclose,
                     acc_v[0, :], acc_v[1, :], acc_v[2, :], acc_v[3, :], s))
            acc_v[0, :], acc_v[1, :] = sm0, sm1
            acc_v[2, :], acc_v[3, :] = mx0, mx1
            acc_v[4, :] = cnt
            g = jnp.where(live, s + _CH, g)
            return g, done, active, prev, nclose

        n_mid = (re - g0) // _CH
        active, prev, nclose = lax.fori_loop(
            0, n_mid, chunk_mid,
            (jnp.bool_(False), jnp.int32(-1), jnp.int32(0)))

        carry = (g0 + n_mid * _CH, jnp.bool_(False), active, prev, nclose)
        g, done, active, prev, nclose = lax.fori_loop(
            0, _N // _CH, chunk, carry)

        @pl.when(active & ~done)
        def _():
            emit(nclose, prev, acc_v[4, :],
                 acc_v[0, :], acc_v[1, :], acc_v[2, :], acc_v[3, :])

        nclose = nclose + (active & ~done).astype(jnp.int32)

        def drain(i, c):
            pltpu.make_async_copy(tab_hbm.at[pl.ds(0, _DH)],
                                  ring_v.at[pl.ds(0, _DH)], sem).wait()
            return c

        lax.fori_loop(0, jnp.minimum(nclose, _NR), drain, jnp.int32(0))

    return k(ids, feat)


def _gather(ids, tab):
    ngrp = _N // 16

    @functools.partial(
        pl.kernel,
        out_type=jax.ShapeDtypeStruct((_N, _DH), jnp.float32),
        mesh=_sc_mesh(),
        scratch_types=[
            pltpu.VMEM((_OC,), jnp.int32),
            pltpu.VMEM((_OC, _DH), jnp.float32),
            pltpu.VMEM((_TW * _DH,), jnp.float32),
            pltpu.VMEM((_DH,), jnp.float32),
        ],
    )
    def k(ids_hbm, tab_hbm, out_hbm, ids_v, ost_v, win_v, side_v):
        wid = lax.axis_index("s") * 2 + lax.axis_index("c")
        ra = ((wid * ngrp) // _NW) * 16
        rb = (((wid + 1) * ngrp) // _NW) * 16
        nch = (rb - ra + _OC - 1) // _OC

        def group(gi, wb):
            loc = gi * 16
            idv = ids_v[pl.ds(loc, 16)]
            lo, hi = idv[0], idv[15]
            trig = (hi >= wb + _TW) | (lo < wb)
            wb = jnp.where(trig, jnp.minimum(lo, _NSEG - _TW), wb)

            @pl.when(trig)
            def _():
                off = pl.multiple_of(wb * _DH, 8)
                pltpu.sync_copy(tab_hbm.at[pl.ds(off, _TW * _DH)], win_v)

            for j in range(16):
                id_j = idv[j]
                d = jnp.clip(id_j - wb, 0, _TW - 1)
                off = pl.multiple_of(d * _DH, 8)
                ost_v[loc + j, pl.ds(0, 16)] = win_v[pl.ds(off, 16)]
                ost_v[loc + j, pl.ds(16, 16)] = win_v[pl.ds(off + 16, 16)]

            @pl.when((hi - wb >= _TW) | (lo < wb))
            def _():
                for j in range(16):
                    id_j = idv[j]

                    @pl.when((id_j - wb >= _TW) | (id_j < wb))
                    def _():
                        toff = pl.multiple_of(id_j * _DH, 8)
                        pltpu.sync_copy(tab_hbm.at[pl.ds(toff, _DH)], side_v)
                        ost_v[loc + j, pl.ds(0, 16)] = side_v[pl.ds(0, 16)]
                        ost_v[loc + j, pl.ds(16, 16)] = side_v[pl.ds(16, 16)]
            return wb

        def chunk(c, wb):
            s = pl.multiple_of(jnp.minimum(ra + c * _OC, rb - _OC), 16)
            pltpu.sync_copy(ids_hbm.at[pl.ds(s, _OC)], ids_v)
            wb = lax.fori_loop(0, _OC // 16, group, wb)
            pltpu.sync_copy(ost_v, out_hbm.at[pl.ds(s, _OC), :])
            return wb

        lax.fori_loop(0, nch, chunk, jnp.int32(-_TW))

    return k(ids, tab)


def kernel(inputs, unq_inv, W, b, gamma, beta):
    scale, shift = _fold_bn(inputs, W, b, gamma, beta)
    feat = _feat(inputs, W, scale, shift)
    tab = _seg_pool(unq_inv, feat)
    out2 = _gather(unq_inv, tab)
    return jnp.concatenate([feat, out2], axis=-1)


# speedup vs baseline: 3.2662x; 1.0001x over previous
import functools

import jax
import jax.numpy as jnp
from jax import lax
from jax.experimental import pallas as pl
from jax.experimental.pallas import tpu as pltpu
from jax.experimental.pallas import tpu_sc as plsc

_N = 800000
_DIN = 10
_DH = 32
_NSEG = 50000
_EPS = 1e-3

_NW = 32
_CH = 896
_OC = 448
_TW = 512
_NR = 8
_BR = 8000
_BS = 16000


def _sc_mesh():
    return plsc.VectorSubcoreMesh(core_axis_name="c", subcore_axis_name="s",
                                  num_cores=2, num_subcores=16)


def _stats_body(x_ref, w_ref, b_ref, g_ref, be_ref, sc_ref, sh_ref, s1, s2):
    i = pl.program_id(0)

    @pl.when(i == 0)
    def _():
        s1[...] = jnp.zeros_like(s1)
        s2[...] = jnp.zeros_like(s2)

    x = x_ref[...]
    s1[...] += jnp.sum(x, axis=0, keepdims=True)
    s2[...] += lax.dot_general(x, x, (((0,), (0,)), ((), ())),
                               preferred_element_type=jnp.float32,
                               precision=lax.Precision.HIGHEST)

    @pl.when(i == pl.num_programs(0) - 1)
    def _():
        hi = lax.Precision.HIGHEST
        w = w_ref[...]
        m = s1[...] / _N
        c = s2[...] / _N - lax.dot_general(
            m, m, (((0,), (0,)), ((), ())),
            preferred_element_type=jnp.float32, precision=hi)
        mu = lax.dot_general(m, w, (((1,), (1,)), ((), ())),
                             preferred_element_type=jnp.float32,
                             precision=hi) + b_ref[...]
        wc = lax.dot_general(w, c, (((1,), (0,)), ((), ())),
                             preferred_element_type=jnp.float32,
                             precision=hi)
        ones = jnp.ones((1, _DIN), jnp.float32)
        var = lax.dot_general(ones, wc * w, (((1,), (1,)), ((), ())),
                              preferred_element_type=jnp.float32,
                              precision=hi)
        scale = g_ref[...] * lax.rsqrt(var + _EPS)
        sc_ref[...] = scale
        sh_ref[...] = (b_ref[...] - mu) * scale + be_ref[...]


def _fold_bn(inputs, W, b, gamma, beta):
    nb = _N // _BS
    full = pl.BlockSpec((_DH, _DIN), lambda i: (0, 0))
    row = pl.BlockSpec((1, _DH), lambda i: (0, 0))
    return pl.pallas_call(
        _stats_body,
        grid=(nb,),
        in_specs=[pl.BlockSpec((_BS, _DIN), lambda i: (i, 0)), full, row, row, row],
        out_specs=[row, row],
        out_shape=[jax.ShapeDtypeStruct((1, _DH), jnp.float32),
                   jax.ShapeDtypeStruct((1, _DH), jnp.float32)],
        scratch_shapes=[pltpu.VMEM((1, _DIN), jnp.float32),
                        pltpu.VMEM((_DIN, _DIN), jnp.float32)],
    )(inputs, W, b.reshape(1, _DH), gamma.reshape(1, _DH), beta.reshape(1, _DH))


def _feat_body(x_ref, w_ref, sc_ref, sh_ref, o_ref):
    x = x_ref[...]
    y = lax.dot_general(x, w_ref[...], (((1,), (1,)), ((), ())),
                        preferred_element_type=jnp.float32,
                        precision=lax.Precision.HIGHEST)
    o_ref[...] = jnp.maximum(y * sc_ref[...] + sh_ref[...], 0.0)


def _feat(inputs, w, scale, shift):
    nb = _N // _BR
    row = pl.BlockSpec((1, _DH), lambda i: (0, 0))
    return pl.pallas_call(
        _feat_body,
        grid=(nb,),
        in_specs=[pl.BlockSpec((_BR, _DIN), lambda i: (i, 0)),
                  pl.BlockSpec((_DH, _DIN), lambda i: (0, 0)), row, row],
        out_specs=pl.BlockSpec((_BR, _DH), lambda i: (i, 0)),
        out_shape=jax.ShapeDtypeStruct((_N, _DH), jnp.float32),
    )(inputs, w, scale, shift)


def _seg_pool(ids, feat):
    ngrp = _N // 16

    @functools.partial(
        pl.kernel,
        out_type=jax.ShapeDtypeStruct((_NSEG * _DH,), jnp.float32),
        mesh=_sc_mesh(),
        scratch_types=[
            pltpu.VMEM((_CH,), jnp.int32),
            pltpu.VMEM((_CH, _DH), jnp.float32),
            pltpu.VMEM((_NR * _DH,), jnp.float32),
            pltpu.VMEM((5, 16), jnp.float32),
            pltpu.VMEM((16 * 96,), jnp.float32),
            pltpu.SemaphoreType.DMA,
        ],
    )
    def k(ids_hbm, feat_hbm, tab_hbm, ids_v, feat_v, ring_v, acc_v, pend_v,
          sem):
        wid = lax.axis_index("s") * 2 + lax.axis_index("c")
        rs = ((wid * ngrp) // _NW) * 16
        re = (((wid + 1) * ngrp) // _NW) * 16
        g0 = jnp.maximum(rs - 16, 0)

        zero = jnp.zeros((16,), jnp.float32)

        def emit(nclose, prev, cnt, sm0, sm1, mx0, mx1):
            slot = pl.multiple_of((nclose & (_NR - 1)) * _DH, 8)

            @pl.when(nclose >= _NR)
            def _():
                pltpu.make_async_copy(
                    tab_hbm.at[pl.ds(0, _DH)], ring_v.at[pl.ds(slot, _DH)],
                    sem).wait()

            inv = 1.0 / cnt
            ring_v[pl.ds(slot, 16)] = 0.5 * (mx0 + sm0 * inv)
            ring_v[pl.ds(slot + 16, 16)] = 0.5 * (mx1 + sm1 * inv)
            off = pl.multiple_of(prev * _DH, 8)
            pltpu.async_copy(ring_v.at[pl.ds(slot, _DH)],
                             tab_hbm.at[pl.ds(off, _DH)], sem)

        def group(gi, carry):
            done, active, prev, cnt, nclose, sm0, sm1, mx0, mx1, s = carry
            loc = gi * 16
            idv = ids_v[pl.ds(loc, 16)]
            for j in range(16):
                g_row = s + loc + j
                id_j = idv[j]
                r0 = feat_v[loc + j, pl.ds(0, 16)]
                r1 = feat_v[loc + j, pl.ds(16, 16)]
                is_new = (id_j != prev) & ~done
                close = is_new & active
                stop = is_new & (g_row >= re)

                @pl.when(close)
                def _():
                    emit(nclose, prev, cnt, sm0, sm1, mx0, mx1)

                nclose = jnp.where(close, nclose + 1, nclose)
                done = done | stop
                active = (active | (is_new & (g_row >= rs))) & ~done
                sm0 = jnp.where(is_new, r0, sm0 + r0)
                sm1 = jnp.where(is_new, r1, sm1 + r1)
                mx0 = jnp.where(is_new, r0, jnp.maximum(mx0, r0))
                mx1 = jnp.where(is_new, r1, jnp.maximum(mx1, r1))
                cnt = jnp.where(is_new, 1.0, cnt + 1.0)
                prev = id_j
            return done, active, prev, cnt, nclose, sm0, sm1, mx0, mx1, s

        def group_mid(gi, carry):
            active, prev, cnt, nclose, sm0, sm1, mx0, mx1, s = carry
            loc = gi * 16
            idv = ids_v[pl.ds(loc, 16)]
            pc = jnp.int32(0)
            for j in range(16):
                id_j = idv[j]
                r0 = feat_v[loc + j, pl.ds(0, 16)]
                r1 = feat_v[loc + j, pl.ds(16, 16)]
                is_new = id_j != prev
                close = is_new & active

                @pl.when(close)
                def _():
                    base = pc * 96
                    pend_v[pl.ds(base, 16)] = sm0
                    pend_v[pl.ds(base + 16, 16)] = sm1
                    pend_v[pl.ds(base + 32, 16)] = mx0
                    pend_v[pl.ds(base + 48, 16)] = mx1
                    pend_v[pl.ds(base + 64, 16)] = cnt
                    pend_v[pl.ds(base + 80, 16)] = jnp.broadcast_to(
                        prev.astype(jnp.float32), (16,))

                pc = jnp.where(close, pc + 1, pc)
                active = active | (is_new & ((s + loc + j) >= rs))
                sm0 = jnp.where(is_new, r0, sm0 + r0)
                sm1 = jnp.where(is_new, r1, sm1 + r1)
                mx0 = jnp.where(is_new, r0, jnp.maximum(mx0, r0))
                mx1 = jnp.where(is_new, r1, jnp.maximum(mx1, r1))
                cnt = jnp.where(is_new, 1.0, cnt + 1.0)
                prev = id_j

            def drain_pend(i, ncl):
                b = i * 96
                pid = pend_v[pl.ds(b + 80, 16)][0].astype(jnp.int32)
                emit(ncl, pid, pend_v[pl.ds(b + 64, 16)],
                     pend_v[pl.ds(b, 16)], pend_v[pl.ds(b + 16, 16)],
                     pend_v[pl.ds(b + 32, 16)], pend_v[pl.ds(b + 48, 16)])
                return ncl + 1

            nclose = lax.fori_loop(0, pc, drain_pend, nclose)
            return active, prev, cnt, nclose, sm0, sm1, mx0, mx1, s

        def chunk_mid(k, carry):
            active, prev, nclose = carry
            s = pl.multiple_of(g0 + k * _CH, 16)
            pltpu.sync_copy(ids_hbm.at[pl.ds(s, _CH)], ids_v)
            pltpu.sync_copy(feat_hbm.at[pl.ds(s, _CH), :], feat_v)
            active, prev, cnt, nclose, sm0, sm1, mx0, mx1, _ = lax.fori_loop(
                0, _CH // 16, group_mid,
                (active, prev, acc_v[4, :], nclose,
                 acc_v[0, :], acc_v[1, :], acc_v[2, :], acc_v[3, :], s))
            acc_v[0, :], acc_v[1, :] = sm0, sm1
            acc_v[2, :], acc_v[3, :] = mx0, mx1
            acc_v[4, :] = cnt
            return active, prev, nclose

        def chunk(_, carry):
            g, done, active, prev, nclose = carry
            live = (~done) & (g < _N)
            s = pl.multiple_of(jnp.minimum(g, _N - _CH), 16)

            @pl.when(live)
            def _():
                pltpu.sync_copy(ids_hbm.at[pl.ds(s, _CH)], ids_v)
                pltpu.sync_copy(feat_hbm.at[pl.ds(s, _CH), :], feat_v)

            lo = jnp.where(live, (g - s) // 16, _CH // 16)
            done, active, prev, cnt, nclose, sm0, sm1, mx0, mx1, _ = \
                lax.fori_loop(
                    lo, _CH // 16, group,
                    (done, active, prev, acc_v[4, :], nclose,
                     acc_v[0, :], acc_v[1, :], acc_v[2, :], acc_v[3, :], s))
            acc_v[0, :], acc_v[1, :] = sm0, sm1
            acc_v[2, :], acc_v[3, :] = mx0, mx1
            acc_v[4, :] = cnt
            g = jnp.where(live, s + _CH, g)
            return g, done, active, prev, nclose

        n_mid = (re - g0) // _CH
        active, prev, nclose = lax.fori_loop(
            0, n_mid, chunk_mid,
            (jnp.bool_(False), jnp.int32(-1), jnp.int32(0)))

        carry = (g0 + n_mid * _CH, jnp.bool_(False), active, prev, nclose)
        g, done, active, prev, nclose = lax.fori_loop(
            0, _N // _CH, chunk, carry)

        @pl.when(active & ~done)
        def _():
            emit(nclose, prev, acc_v[4, :],
                 acc_v[0, :], acc_v[1, :], acc_v[2, :], acc_v[3, :])

        nclose = nclose + (active & ~done).astype(jnp.int32)

        def drain(i, c):
            pltpu.make_async_copy(tab_hbm.at[pl.ds(0, _DH)],
                                  ring_v.at[pl.ds(0, _DH)], sem).wait()
            return c

        lax.fori_loop(0, jnp.minimum(nclose, _NR), drain, jnp.int32(0))

    return k(ids, feat)


def _gather(ids, tab):
    ngrp = _N // 16

    @functools.partial(
        pl.kernel,
        out_type=jax.ShapeDtypeStruct((_N, _DH), jnp.float32),
        mesh=_sc_mesh(),
        scratch_types=[
            pltpu.VMEM((_OC,), jnp.int32),
            pltpu.VMEM((_OC, _DH), jnp.float32),
            pltpu.VMEM((_TW * _DH,), jnp.float32),
            pltpu.VMEM((_DH,), jnp.float32),
        ],
    )
    def k(ids_hbm, tab_hbm, out_hbm, ids_v, ost_v, win_v, side_v):
        wid = lax.axis_index("s") * 2 + lax.axis_index("c")
        ra = ((wid * ngrp) // _NW) * 16
        rb = (((wid + 1) * ngrp) // _NW) * 16
        nch = (rb - ra + _OC - 1) // _OC

        def group(gi, wb):
            loc = gi * 16
            idv = ids_v[pl.ds(loc, 16)]
            lo, hi = idv[0], idv[15]
            trig = (hi >= wb + _TW) | (lo < wb)
            wb = jnp.where(trig, jnp.minimum(lo, _NSEG - _TW), wb)

            @pl.when(trig)
            def _():
                off = pl.multiple_of(wb * _DH, 8)
                pltpu.sync_copy(tab_hbm.at[pl.ds(off, _TW * _DH)], win_v)

            for j in range(16):
                id_j = idv[j]
                d = jnp.clip(id_j - wb, 0, _TW - 1)
                off = pl.multiple_of(d * _DH, 8)
                ost_v[loc + j, pl.ds(0, 16)] = win_v[pl.ds(off, 16)]
                ost_v[loc + j, pl.ds(16, 16)] = win_v[pl.ds(off + 16, 16)]

            def patch(_, c):
                for j in range(16):
                    id_j = idv[j]

                    @pl.when((id_j - wb >= _TW) | (id_j < wb))
                    def _():
                        toff = pl.multiple_of(id_j * _DH, 8)
                        pltpu.sync_copy(tab_hbm.at[pl.ds(toff, _DH)], side_v)
                        ost_v[loc + j, pl.ds(0, 16)] = side_v[pl.ds(0, 16)]
                        ost_v[loc + j, pl.ds(16, 16)] = side_v[pl.ds(16, 16)]
                return c

            nfall = ((hi - wb >= _TW) | (lo < wb)).astype(jnp.int32)
            lax.fori_loop(0, nfall, patch, jnp.int32(0))
            return wb

        def chunk(c, wb):
            s = pl.multiple_of(jnp.minimum(ra + c * _OC, rb - _OC), 16)
            pltpu.sync_copy(ids_hbm.at[pl.ds(s, _OC)], ids_v)
            wb = lax.fori_loop(0, _OC // 16, group, wb)
            pltpu.sync_copy(ost_v, out_hbm.at[pl.ds(s, _OC), :])
            return wb

        lax.fori_loop(0, nch, chunk, jnp.int32(-_TW))

    return k(ids, tab)


def kernel(inputs, unq_inv, W, b, gamma, beta):
    scale, shift = _fold_bn(inputs, W, b, gamma, beta)
    feat = _feat(inputs, W, scale, shift)
    tab = _seg_pool(unq_inv, feat)
    out2 = _gather(unq_inv, tab)
    return jnp.concatenate([feat, out2], axis=-1)


# speedup vs baseline: 3.2666x; 1.0001x over previous
import functools

import jax
import jax.numpy as jnp
from jax import lax
from jax.experimental import pallas as pl
from jax.experimental.pallas import tpu as pltpu
from jax.experimental.pallas import tpu_sc as plsc

_N = 800000
_DIN = 10
_DH = 32
_NSEG = 50000
_EPS = 1e-3

_NW = 32
_CH = 896
_OC = 448
_TW = 512
_NR = 8
_BR = 8000
_BS = 16000


def _sc_mesh():
    return plsc.VectorSubcoreMesh(core_axis_name="c", subcore_axis_name="s",
                                  num_cores=2, num_subcores=16)


def _stats_body(x_ref, w_ref, b_ref, g_ref, be_ref, sc_ref, sh_ref, s1, s2):
    i = pl.program_id(0)

    @pl.when(i == 0)
    def _():
        s1[...] = jnp.zeros_like(s1)
        s2[...] = jnp.zeros_like(s2)

    x = x_ref[...]
    s1[...] += jnp.sum(x, axis=0, keepdims=True)
    s2[...] += lax.dot_general(x, x, (((0,), (0,)), ((), ())),
                               preferred_element_type=jnp.float32,
                               precision=lax.Precision.HIGHEST)

    @pl.when(i == pl.num_programs(0) - 1)
    def _():
        hi = lax.Precision.HIGHEST
        w = w_ref[...]
        m = s1[...] / _N
        c = s2[...] / _N - lax.dot_general(
            m, m, (((0,), (0,)), ((), ())),
            preferred_element_type=jnp.float32, precision=hi)
        mu = lax.dot_general(m, w, (((1,), (1,)), ((), ())),
                             preferred_element_type=jnp.float32,
                             precision=hi) + b_ref[...]
        wc = lax.dot_general(w, c, (((1,), (0,)), ((), ())),
                             preferred_element_type=jnp.float32,
                             precision=hi)
        ones = jnp.ones((1, _DIN), jnp.float32)
        var = lax.dot_general(ones, wc * w, (((1,), (1,)), ((), ())),
                              preferred_element_type=jnp.float32,
                              precision=hi)
        scale = g_ref[...] * lax.rsqrt(var + _EPS)
        sc_ref[...] = scale
        sh_ref[...] = (b_ref[...] - mu) * scale + be_ref[...]


def _fold_bn(inputs, W, b, gamma, beta):
    nb = _N // _BS
    full = pl.BlockSpec((_DH, _DIN), lambda i: (0, 0))
    row = pl.BlockSpec((1, _DH), lambda i: (0, 0))
    return pl.pallas_call(
        _stats_body,
        grid=(nb,),
        in_specs=[pl.BlockSpec((_BS, _DIN), lambda i: (i, 0)), full, row, row, row],
        out_specs=[row, row],
        out_shape=[jax.ShapeDtypeStruct((1, _DH), jnp.float32),
                   jax.ShapeDtypeStruct((1, _DH), jnp.float32)],
        scratch_shapes=[pltpu.VMEM((1, _DIN), jnp.float32),
                        pltpu.VMEM((_DIN, _DIN), jnp.float32)],
    )(inputs, W, b.reshape(1, _DH), gamma.reshape(1, _DH), beta.reshape(1, _DH))


def _feat_body(x_ref, w_ref, sc_ref, sh_ref, o_ref):
    x = x_ref[...]
    y = lax.dot_general(x, w_ref[...], (((1,), (1,)), ((), ())),
                        preferred_element_type=jnp.float32,
                        precision=lax.Precision.HIGHEST)
    o_ref[...] = jnp.maximum(y * sc_ref[...] + sh_ref[...], 0.0)


def _feat(inputs, w, scale, shift):
    nb = _N // _BR
    row = pl.BlockSpec((1, _DH), lambda i: (0, 0))
    return pl.pallas_call(
        _feat_body,
        grid=(nb,),
        in_specs=[pl.BlockSpec((_BR, _DIN), lambda i: (i, 0)),
                  pl.BlockSpec((_DH, _DIN), lambda i: (0, 0)), row, row],
        out_specs=pl.BlockSpec((_BR, _DH), lambda i: (i, 0)),
        out_shape=jax.ShapeDtypeStruct((_N, _DH), jnp.float32),
    )(inputs, w, scale, shift)


def _seg_pool(ids, feat):
    ngrp = _N // 16

    @functools.partial(
        pl.kernel,
        out_type=jax.ShapeDtypeStruct((_NSEG * _DH,), jnp.float32),
        mesh=_sc_mesh(),
        scratch_types=[
            pltpu.VMEM((_CH,), jnp.int32),
            pltpu.VMEM((_CH, _DH), jnp.float32),
            pltpu.VMEM((_NR * _DH,), jnp.float32),
            pltpu.VMEM((5, 16), jnp.float32),
            pltpu.VMEM((16 * 96,), jnp.float32),
            pltpu.SemaphoreType.DMA,
        ],
    )
    def k(ids_hbm, feat_hbm, tab_hbm, ids_v, feat_v, ring_v, acc_v, pend_v,
          sem):
        wid = lax.axis_index("s") * 2 + lax.axis_index("c")
        rs = ((wid * ngrp) // _NW) * 16
        re = (((wid + 1) * ngrp) // _NW) * 16
        g0 = jnp.maximum(rs - 16, 0)

        def emit(nclose, prev, cnt, sm0, sm1, mx0, mx1):
            slot = pl.multiple_of((nclose & (_NR - 1)) * _DH, 8)

            @pl.when(nclose >= _NR)
            def _():
                pltpu.make_async_copy(
                    tab_hbm.at[pl.ds(0, _DH)], ring_v.at[pl.ds(slot, _DH)],
                    sem).wait()

            inv = 1.0 / cnt
            ring_v[pl.ds(slot, 16)] = 0.5 * (mx0 + sm0 * inv)
            ring_v[pl.ds(slot + 16, 16)] = 0.5 * (mx1 + sm1 * inv)
            off = pl.multiple_of(prev * _DH, 8)
            pltpu.async_copy(ring_v.at[pl.ds(slot, _DH)],
                             tab_hbm.at[pl.ds(off, _DH)], sem)

        def group(gi, carry):
            done, active, prev, cnt, nclose, sm0, sm1, mx0, mx1, s = carry
            loc = gi * 16
            idv = ids_v[pl.ds(loc, 16)]
            for j in range(16):
                g_row = s + loc + j
                id_j = idv[j]
                r0 = feat_v[loc + j, pl.ds(0, 16)]
                r1 = feat_v[loc + j, pl.ds(16, 16)]
                is_new = (id_j != prev) & ~done
                close = is_new & active
                stop = is_new & (g_row >= re)

                @pl.when(close)
                def _():
                    emit(nclose, prev, cnt, sm0, sm1, mx0, mx1)

                nclose = jnp.where(close, nclose + 1, nclose)
                done = done | stop
                active = (active | (is_new & (g_row >= rs))) & ~done
                sm0 = jnp.where(is_new, r0, sm0 + r0)
                sm1 = jnp.where(is_new, r1, sm1 + r1)
                mx0 = jnp.where(is_new, r0, jnp.maximum(mx0, r0))
                mx1 = jnp.where(is_new, r1, jnp.maximum(mx1, r1))
                cnt = jnp.where(is_new, 1.0, cnt + 1.0)
                prev = id_j
            return done, active, prev, cnt, nclose, sm0, sm1, mx0, mx1, s

        def group_mid(gi, carry):
            active, prev, cnt, nclose, sm0, sm1, mx0, mx1, s = carry
            loc = gi * 16
            idv = ids_v[pl.ds(loc, 16)]
            pc = jnp.int32(0)
            for j in range(16):
                id_j = idv[j]
                r0 = feat_v[loc + j, pl.ds(0, 16)]
                r1 = feat_v[loc + j, pl.ds(16, 16)]
                is_new = id_j != prev
                close = is_new & active

                @pl.when(close)
                def _():
                    base = pc * 96
                    pend_v[pl.ds(base, 16)] = sm0
                    pend_v[pl.ds(base + 16, 16)] = sm1
                    pend_v[pl.ds(base + 32, 16)] = mx0
                    pend_v[pl.ds(base + 48, 16)] = mx1
                    pend_v[pl.ds(base + 64, 16)] = cnt
                    pend_v[pl.ds(base + 80, 16)] = jnp.broadcast_to(
                        prev.astype(jnp.float32), (16,))

                pc = jnp.where(close, pc + 1, pc)
                active = active | (is_new & ((s + loc + j) >= rs))
                sm0 = jnp.where(is_new, r0, sm0 + r0)
                sm1 = jnp.where(is_new, r1, sm1 + r1)
                mx0 = jnp.where(is_new, r0, jnp.maximum(mx0, r0))
                mx1 = jnp.where(is_new, r1, jnp.maximum(mx1, r1))
                cnt = jnp.where(is_new, 1.0, cnt + 1.0)
                prev = id_j

            def drain_pend(i, ncl):
                b = i * 96
                pid = pend_v[pl.ds(b + 80, 16)][0].astype(jnp.int32)
                emit(ncl, pid, pend_v[pl.ds(b + 64, 16)],
                     pend_v[pl.ds(b, 16)], pend_v[pl.ds(b + 16, 16)],
                     pend_v[pl.ds(b + 32, 16)], pend_v[pl.ds(b + 48, 16)])
                return ncl + 1

            nclose = lax.fori_loop(0, pc, drain_pend, nclose)
            return active, prev, cnt, nclose, sm0, sm1, mx0, mx1, s

        def chunk_mid(k, carry):
            active, prev, nclose = carry
            s = pl.multiple_of(g0 + k * _CH, 16)
            pltpu.sync_copy(ids_hbm.at[pl.ds(s, _CH)], ids_v)
            pltpu.sync_copy(feat_hbm.at[pl.ds(s, _CH), :], feat_v)
            active, prev, cnt, nclose, sm0, sm1, mx0, mx1, _ = lax.fori_loop(
                0, _CH // 16, group_mid,
                (active, prev, acc_v[4, :], nclose,
                 acc_v[0, :], acc_v[1, :], acc_v[2, :], acc_v[3, :], s))
            acc_v[0, :], acc_v[1, :] = sm0, sm1
            acc_v[2, :], acc_v[3, :] = mx0, mx1
            acc_v[4, :] = cnt
            return active, prev, nclose

        def chunk(_, carry):
            g, done, active, prev, nclose = carry
            live = (~done) & (g < _N)
            s = pl.multiple_of(jnp.minimum(g, _N - _CH), 16)

            @pl.when(live)
            def _():
                pltpu.sync_copy(ids_hbm.at[pl.ds(s, _CH)], ids_v)
                pltpu.sync_copy(feat_hbm.at[pl.ds(s, _CH), :], feat_v)

            lo = jnp.where(live, (g - s) // 16, _CH // 16)
            done, active, prev, cnt, nclose, sm0, sm1, mx0, mx1, _ = \
                lax.fori_loop(
                    lo, _CH // 16, group,
                    (done, active, prev, acc_v[4, :], nclose,
                     acc_v[0, :], acc_v[1, :], acc_v[2, :], acc_v[3, :], s))
            acc_v[0, :], acc_v[1, :] = sm0, sm1
            acc_v[2, :], acc_v[3, :] = mx0, mx1
            acc_v[4, :] = cnt
            g = jnp.where(live, s + _CH, g)
            return g, done, active, prev, nclose

        n_mid = (re - g0) // _CH
        active, prev, nclose = lax.fori_loop(
            0, n_mid, chunk_mid,
            (jnp.bool_(False), jnp.int32(-1), jnp.int32(0)))

        carry = (g0 + n_mid * _CH, jnp.bool_(False), active, prev, nclose)
        g, done, active, prev, nclose = lax.fori_loop(
            0, _N // _CH, chunk, carry)

        @pl.when(active & ~done)
        def _():
            emit(nclose, prev, acc_v[4, :],
                 acc_v[0, :], acc_v[1, :], acc_v[2, :], acc_v[3, :])

        nclose = nclose + (active & ~done).astype(jnp.int32)

        def drain(i, c):
            pltpu.make_async_copy(tab_hbm.at[pl.ds(0, _DH)],
                                  ring_v.at[pl.ds(0, _DH)], sem).wait()
            return c

        lax.fori_loop(0, jnp.minimum(nclose, _NR), drain, jnp.int32(0))

    return k(ids, feat)


def _gather(ids, tab):
    ngrp = _N // 16

    @functools.partial(
        pl.kernel,
        out_type=jax.ShapeDtypeStruct((_N, _DH), jnp.float32),
        mesh=_sc_mesh(),
        scratch_types=[
            pltpu.VMEM((_OC,), jnp.int32),
            pltpu.VMEM((_OC, _DH), jnp.float32),
            pltpu.VMEM((_TW * _DH,), jnp.float32),
            pltpu.VMEM((_DH,), jnp.float32),
        ],
    )
    def k(ids_hbm, tab_hbm, out_hbm, ids_v, ost_v, win_v, side_v):
        wid = lax.axis_index("s") * 2 + lax.axis_index("c")
        ra = ((wid * ngrp) // _NW) * 16
        rb = (((wid + 1) * ngrp) // _NW) * 16
        nch = (rb - ra + _OC - 1) // _OC

        def group(gi, wb):
            loc = gi * 16
            idv = ids_v[pl.ds(loc, 16)]
            lo, hi = idv[0], idv[15]
            trig = (hi >= wb + _TW) | (lo < wb)
            wb = jnp.where(trig, jnp.minimum(lo, _NSEG - _TW), wb)

            @pl.when(trig)
            def _():
                off = pl.multiple_of(wb * _DH, 8)
                pltpu.sync_copy(tab_hbm.at[pl.ds(off, _TW * _DH)], win_v)

            for j in range(16):
                id_j = idv[j]
                d = jnp.clip(id_j - wb, 0, _TW - 1)
                off = pl.multiple_of(d * _DH, 8)
                ost_v[loc + j, pl.ds(0, 16)] = win_v[pl.ds(off, 16)]
                ost_v[loc + j, pl.ds(16, 16)] = win_v[pl.ds(off + 16, 16)]

            def patch(_, c):
                for j in range(16):
                    id_j = idv[j]

                    @pl.when((id_j - wb >= _TW) | (id_j < wb))
                    def _():
                        toff = pl.multiple_of(id_j * _DH, 8)
                        pltpu.sync_copy(tab_hbm.at[pl.ds(toff, _DH)], side_v)
                        ost_v[loc + j, pl.ds(0, 16)] = side_v[pl.ds(0, 16)]
                        ost_v[loc + j, pl.ds(16, 16)] = side_v[pl.ds(16, 16)]
                return c

            nfall = ((hi - wb >= _TW) | (lo < wb)).astype(jnp.int32)
            lax.fori_loop(0, nfall, patch, jnp.int32(0))
            return wb

        def chunk(c, wb):
            s = pl.multiple_of(jnp.minimum(ra + c * _OC, rb - _OC), 16)
            pltpu.sync_copy(ids_hbm.at[pl.ds(s, _OC)], ids_v)
            wb = lax.fori_loop(0, _OC // 16, group, wb)
            pltpu.sync_copy(ost_v, out_hbm.at[pl.ds(s, _OC), :])
            return wb

        lax.fori_loop(0, nch, chunk, jnp.int32(-_TW))

    return k(ids, tab)


def kernel(inputs, unq_inv, W, b, gamma, beta):
    scale, shift = _fold_bn(inputs, W, b, gamma, beta)
    feat = _feat(inputs, W, scale, shift)
    tab = _seg_pool(unq_inv, feat)
    out2 = _gather(unq_inv, tab)
    return jnp.concatenate([feat, out2], axis=-1)
